# baseline (speedup 1.0000x reference)
import functools
import math

import jax
import jax.numpy as jnp
from jax import lax
from jax.experimental import pallas as pl
from jax.experimental.pallas import tpu as pltpu
from jax.experimental.pallas import tpu_sc as plsc

N = 10000
P = 320000
F = 128
H = 4
FH = F // H
M = 16

NC = 2
NS = 16
NW = NC * NS
EPW = P // NW
C = 80
NCHUNK = EPW // C
NPAD = 10240
ROWS_PER_TILE = NPAD // NS


def _prep_body(x_ref, wq_ref, wk_ref, q_ref, k_ref):
    dn = (((1,), (1,)), ((), ()))
    for h in range(H):
        xh = x_ref[:, h * FH:(h + 1) * FH]
        q_ref[:, h * FH:(h + 1) * FH] = lax.dot_general(
            xh, wq_ref[h], dn, preferred_element_type=jnp.float32)
        k_ref[:, h * FH:(h + 1) * FH] = lax.dot_general(
            xh, wk_ref[h], dn, preferred_element_type=jnp.float32)


_RSQRT_FH = 1.0 / math.sqrt(FH)


_GDN = lax.GatherDimensionNumbers(offset_dims=(), collapsed_slice_dims=(0,),
                                  start_index_map=(0,))


def _shuffle(v, perm):
    return lax.gather(v, perm[:, None], dimension_numbers=_GDN,
                      slice_sizes=(1,), mode=lax.GatherScatterMode.PROMISE_IN_BOUNDS)


def _splat_sum(v, iota):
    for sft in (1, 2, 4, 8):
        v = v + _shuffle(v, jnp.bitwise_xor(iota, sft))
    return v


def _rep_from_heads(hs, iota):
    lo8 = iota < 8
    a = [h + _shuffle(h, jnp.bitwise_xor(iota, 8)) for h in hs]
    ab = jnp.where(lo8, a[0], a[1])
    cd = jnp.where(lo8, a[2], a[3])
    for sft in (4, 2, 1):
        perm = jnp.bitwise_xor(iota, sft)
        ab = ab + _shuffle(ab, perm)
        cd = cd + _shuffle(cd, perm)
    pat_ab = jnp.where(iota < 1, 0, 8)
    pat_cd = jnp.where(iota < 9, 0, 8)
    return jnp.where(iota < 4, _shuffle(ab, pat_ab), _shuffle(cd, pat_cd))
def _sc_edges_body(q_hbm, k_hbm, w_hbm, sph_hbm, phir_hbm, phic_hbm,
                   idxi_hbm, idxj_hbm,
                   out_hbm,
                   acc_sh,
                   idxi_v, idxj_v, q_v, k_v, w_v, sph_v, phir_v, phic_v,
                   scl_v, contrib_v,
                   slab_v, sem_idx, sem_dat, sem_sc):
    cid = lax.axis_index("c")
    sid = lax.axis_index("s")
    wid = sid * NC + cid

    def _zero_row(r, _):
        slab_v[r, :] = jnp.zeros((M,), jnp.float32)
        return 0
    lax.fori_loop(0, ROWS_PER_TILE, _zero_row, 0)
    pltpu.sync_copy(slab_v, acc_sh.at[pl.ds(sid * ROWS_PER_TILE, ROWS_PER_TILE)])
    plsc.subcore_barrier()

    iota = lax.iota(jnp.int32, 16)

    def _base(g):
        return wid * EPW + g * C

    def issue_idx(g, bi):
        pltpu.async_copy(idxi_hbm.at[pl.ds(_base(g), C)], idxi_v[bi], sem_idx[bi])
        pltpu.async_copy(idxj_hbm.at[pl.ds(_base(g), C)], idxj_v[bi], sem_idx[bi])

    def wait_idx(bi):
        pltpu.make_async_copy(idxi_hbm.at[pl.ds(0, C)], idxi_v[bi], sem_idx[bi]).wait()
        pltpu.make_async_copy(idxj_hbm.at[pl.ds(0, C)], idxj_v[bi], sem_idx[bi]).wait()

    def issue_data(g, bi, bd):
        pltpu.async_copy(q_hbm.at[idxi_v[bi]], q_v[bd], sem_dat[bd])
        pltpu.async_copy(k_hbm.at[idxj_v[bi]], k_v[bd], sem_dat[bd])
        pltpu.async_copy(w_hbm.at[pl.ds(_base(g), C)], w_v[bd], sem_dat[bd])
        pltpu.async_copy(sph_hbm.at[pl.ds(_base(g) * M // F, C * M // F)],
                         sph_v[bd], sem_dat[bd])
        pltpu.async_copy(phir_hbm.at[pl.ds(_base(g), C)], phir_v[bd], sem_dat[bd])
        pltpu.async_copy(phic_hbm.at[pl.ds(_base(g), C)], phic_v[bd], sem_dat[bd])

    def wait_data(bi, bd):
        pltpu.make_async_copy(q_hbm.at[idxi_v[bi]], q_v[bd], sem_dat[bd]).wait()
        pltpu.make_async_copy(k_hbm.at[idxj_v[bi]], k_v[bd], sem_dat[bd]).wait()
        pltpu.make_async_copy(w_hbm.at[pl.ds(0, C)], w_v[bd], sem_dat[bd]).wait()
        pltpu.make_async_copy(sph_hbm.at[pl.ds(0, C * M // F)], sph_v[bd],
                              sem_dat[bd]).wait()
        pltpu.make_async_copy(phir_hbm.at[pl.ds(0, C)], phir_v[bd], sem_dat[bd]).wait()
        pltpu.make_async_copy(phic_hbm.at[pl.ds(0, C)], phic_v[bd], sem_dat[bd]).wait()

    def issue_scatter(bi, bd):
        pltpu.async_copy(contrib_v[bd], acc_sh.at[idxi_v[bi]], sem_sc[bd], add=True)

    def wait_scatter(bi, bd):
        pltpu.make_async_copy(contrib_v[bd], acc_sh.at[idxi_v[bi]], sem_sc[bd]).wait()

    zero16 = jnp.zeros((16,), jnp.int32)

    def compute(bd):
        for t in range(C // 16):
            s = pl.ds(16 * t, 16)
            scl_v[bd][s] = (phir_v[bd][s] + phic_v[bd][s]) * _RSQRT_FH

        def _edge(e, _):
            hs = []
            for hh in range(H):
                p0 = (q_v[bd][e, pl.ds(32 * hh, 16)]
                      * w_v[bd][e, pl.ds(32 * hh, 16)]
                      * k_v[bd][e, pl.ds(32 * hh, 16)])
                p1 = (q_v[bd][e, pl.ds(32 * hh + 16, 16)]
                      * w_v[bd][e, pl.ds(32 * hh + 16, 16)]
                      * k_v[bd][e, pl.ds(32 * hh + 16, 16)])
                hs.append(p0 + p1)
            rep = _rep_from_heads(hs, iota)
            scale = _shuffle(scl_v[bd][pl.ds(e, 16)], zero16)
            sph_row = sph_v[bd][e // 8, pl.ds((e % 8) * M, 16)]
            contrib_v[bd][e, :] = rep * scale * sph_row
            return 0
        lax.fori_loop(0, C, _edge, 0)

    issue_idx(0, 0)
    issue_idx(1, 1)
    wait_idx(0)
    issue_data(0, 0, 0)

    def _quad(i, _):
        for j in range(4):
            g = 4 * i + j
            bd = j % 2
            bi = j

            @pl.when(g >= 2)
            def _():
                wait_scatter((j - 2) % 4, bd)
            wait_idx((j + 1) % 4)
            issue_data(g + 1, (j + 1) % 4, 1 - bd)
            wait_data(bi, bd)

            @pl.when(g + 2 < NCHUNK)
            def _():
                issue_idx(g + 2, (j + 2) % 4)
            compute(bd)
            issue_scatter(bi, bd)
        return 0
    lax.fori_loop(0, (NCHUNK - 1) // 4, _quad, 0)

    wait_scatter(2, 0)
    wait_data(0, 0)
    compute(0)
    issue_scatter(0, 0)
    wait_scatter(3, 1)
    wait_scatter(0, 0)
    plsc.subcore_barrier()

    row0 = sid * ROWS_PER_TILE
    pltpu.sync_copy(acc_sh.at[pl.ds(row0, ROWS_PER_TILE)], slab_v)
    pltpu.sync_copy(slab_v, out_hbm.at[pl.ds(cid * NPAD + row0, ROWS_PER_TILE)])


def _final_add_body(p_ref, out_ref):
    out_ref[...] = p_ref[:N, :] + p_ref[NPAD:NPAD + N, :]


def kernel(chi, sph_ij, x, w_ij, idx_i, phi_r_cut, phi_chi_cut, idx_j, Wq, Wk):
    del chi
    q, k = pl.pallas_call(
        _prep_body,
        out_shape=(jax.ShapeDtypeStruct((N, F), jnp.float32),
                   jax.ShapeDtypeStruct((N, F), jnp.float32)),
    )(x, Wq, Wk)

    mesh = plsc.VectorSubcoreMesh(core_axis_name="c", subcore_axis_name="s")
    partials = pl.kernel(
        _sc_edges_body,
        mesh=mesh,
        compiler_params=pltpu.CompilerParams(use_tc_tiling_on_sc=False),
        out_type=jax.ShapeDtypeStruct((NC * NPAD, M), jnp.float32),
        scratch_types=[
            pltpu.VMEM_SHARED((NPAD, M), jnp.float32),
            [pltpu.VMEM((C,), jnp.int32) for _ in range(4)],
            [pltpu.VMEM((C,), jnp.int32) for _ in range(4)],
            [pltpu.VMEM((C, F), jnp.float32) for _ in range(2)],
            [pltpu.VMEM((C, F), jnp.float32) for _ in range(2)],
            [pltpu.VMEM((C, F), jnp.float32) for _ in range(2)],
            [pltpu.VMEM((C * M // F, F), jnp.float32) for _ in range(2)],
            [pltpu.VMEM((C,), jnp.float32) for _ in range(2)],
            [pltpu.VMEM((C,), jnp.float32) for _ in range(2)],
            [pltpu.VMEM((C + 16,), jnp.float32) for _ in range(2)],
            [pltpu.VMEM((C, M), jnp.float32) for _ in range(2)],
            pltpu.VMEM((ROWS_PER_TILE, M), jnp.float32),
            [pltpu.SemaphoreType.DMA for _ in range(4)],
            [pltpu.SemaphoreType.DMA for _ in range(2)],
            [pltpu.SemaphoreType.DMA for _ in range(2)],
        ],
    )(q, k, w_ij, sph_ij.reshape(P * M // F, F), phi_r_cut.reshape(P),
      phi_chi_cut, idx_i, idx_j)

    chi_out = pl.pallas_call(
        _final_add_body,
        out_shape=jax.ShapeDtypeStruct((N, M), jnp.float32),
    )(partials)
    return chi_out


# speedup vs baseline: 1.0471x; 1.0471x over previous
import functools
import math

import jax
import jax.numpy as jnp
from jax import lax
from jax.experimental import pallas as pl
from jax.experimental.pallas import tpu as pltpu
from jax.experimental.pallas import tpu_sc as plsc

N = 10000
P = 320000
F = 128
H = 4
FH = F // H
M = 16

NC = 2
NS = 16
NW = NC * NS
EPW = P // NW
C = 80
NCHUNK = EPW // C
NPAD = 10240
ROWS_PER_TILE = NPAD // NS


def _prep_body(x_ref, bq_ref, bk_ref, q_ref, k_ref):
    xv = x_ref[...]
    q_ref[...] = jnp.dot(xv, bq_ref[...], preferred_element_type=jnp.float32)
    k_ref[...] = jnp.dot(xv, bk_ref[...], preferred_element_type=jnp.float32)


_RSQRT_FH = 1.0 / math.sqrt(FH)


_GDN = lax.GatherDimensionNumbers(offset_dims=(), collapsed_slice_dims=(0,),
                                  start_index_map=(0,))


def _shuffle(v, perm):
    return lax.gather(v, perm[:, None], dimension_numbers=_GDN,
                      slice_sizes=(1,), mode=lax.GatherScatterMode.PROMISE_IN_BOUNDS)


def _splat_sum(v, iota):
    for sft in (1, 2, 4, 8):
        v = v + _shuffle(v, jnp.bitwise_xor(iota, sft))
    return v


def _rep_from_heads(hs, iota):
    lo8 = iota < 8
    a = [h + _shuffle(h, jnp.bitwise_xor(iota, 8)) for h in hs]
    ab = jnp.where(lo8, a[0], a[1])
    cd = jnp.where(lo8, a[2], a[3])
    for sft in (4, 2, 1):
        perm = jnp.bitwise_xor(iota, sft)
        ab = ab + _shuffle(ab, perm)
        cd = cd + _shuffle(cd, perm)
    pat_ab = jnp.where(iota < 1, 0, 8)
    pat_cd = jnp.where(iota < 9, 0, 8)
    return jnp.where(iota < 4, _shuffle(ab, pat_ab), _shuffle(cd, pat_cd))
def _sc_edges_body(q_hbm, k_hbm, w_hbm, sph_hbm, phir_hbm, phic_hbm,
                   idxi_hbm, idxj_hbm,
                   out_hbm,
                   acc_sh,
                   idxi_v, idxj_v, q_v, k_v, w_v, sph_v, phir_v, phic_v,
                   scl_v, contrib_v,
                   slab_v, sem_idx, sem_dat, sem_sc):
    cid = lax.axis_index("c")
    sid = lax.axis_index("s")
    wid = sid * NC + cid

    def _zero_row(r, _):
        slab_v[r, :] = jnp.zeros((M,), jnp.float32)
        return 0
    lax.fori_loop(0, ROWS_PER_TILE, _zero_row, 0)
    pltpu.sync_copy(slab_v, acc_sh.at[pl.ds(sid * ROWS_PER_TILE, ROWS_PER_TILE)])
    plsc.subcore_barrier()

    iota = lax.iota(jnp.int32, 16)

    def _base(g):
        return wid * EPW + g * C

    def issue_idx(g, bi):
        pltpu.async_copy(idxi_hbm.at[pl.ds(_base(g), C)], idxi_v[bi], sem_idx[bi])
        pltpu.async_copy(idxj_hbm.at[pl.ds(_base(g), C)], idxj_v[bi], sem_idx[bi])

    def wait_idx(bi):
        pltpu.make_async_copy(idxi_hbm.at[pl.ds(0, C)], idxi_v[bi], sem_idx[bi]).wait()
        pltpu.make_async_copy(idxj_hbm.at[pl.ds(0, C)], idxj_v[bi], sem_idx[bi]).wait()

    def issue_data(g, bi, bd):
        pltpu.async_copy(q_hbm.at[idxi_v[bi]], q_v[bd], sem_dat[bd])
        pltpu.async_copy(k_hbm.at[idxj_v[bi]], k_v[bd], sem_dat[bd])
        pltpu.async_copy(w_hbm.at[pl.ds(_base(g), C)], w_v[bd], sem_dat[bd])
        pltpu.async_copy(sph_hbm.at[pl.ds(_base(g), C)], sph_v[bd], sem_dat[bd])
        pltpu.async_copy(phir_hbm.at[pl.ds(_base(g), C)], phir_v[bd],
                         sem_dat[bd])
        pltpu.async_copy(phic_hbm.at[pl.ds(_base(g), C)], phic_v[bd], sem_dat[bd])

    def wait_data(bi, bd):
        pltpu.make_async_copy(q_hbm.at[idxi_v[bi]], q_v[bd], sem_dat[bd]).wait()
        pltpu.make_async_copy(k_hbm.at[idxj_v[bi]], k_v[bd], sem_dat[bd]).wait()
        pltpu.make_async_copy(w_hbm.at[pl.ds(0, C)], w_v[bd], sem_dat[bd]).wait()
        pltpu.make_async_copy(sph_hbm.at[pl.ds(0, C)], sph_v[bd],
                              sem_dat[bd]).wait()
        pltpu.make_async_copy(phir_hbm.at[pl.ds(0, C)], phir_v[bd],
                              sem_dat[bd]).wait()
        pltpu.make_async_copy(phic_hbm.at[pl.ds(0, C)], phic_v[bd], sem_dat[bd]).wait()

    def issue_scatter(bi, bd):
        pltpu.async_copy(contrib_v[bd], acc_sh.at[idxi_v[bi]], sem_sc[bd], add=True)

    def wait_scatter(bi, bd):
        pltpu.make_async_copy(contrib_v[bd], acc_sh.at[idxi_v[bi]], sem_sc[bd]).wait()

    zero16 = jnp.zeros((16,), jnp.int32)

    def compute(bd):
        for t in range(C // 16):
            s = pl.ds(16 * t, 16)
            scl_v[bd][s] = (phir_v[bd][s] + phic_v[bd][s]) * _RSQRT_FH

        def _edge(e, _):
            hs = []
            for hh in range(H):
                p0 = (q_v[bd][e, pl.ds(32 * hh, 16)]
                      * w_v[bd][e, pl.ds(32 * hh, 16)]
                      * k_v[bd][e, pl.ds(32 * hh, 16)])
                p1 = (q_v[bd][e, pl.ds(32 * hh + 16, 16)]
                      * w_v[bd][e, pl.ds(32 * hh + 16, 16)]
                      * k_v[bd][e, pl.ds(32 * hh + 16, 16)])
                hs.append(_splat_sum(p0 + p1, iota))
            rep = jnp.where(iota < 1, hs[0],
                            jnp.where(iota < 4, hs[1],
                                      jnp.where(iota < 9, hs[2], hs[3])))
            scale = _shuffle(scl_v[bd][pl.ds(e, 16)], zero16)
            contrib_v[bd][e, :] = rep * scale * sph_v[bd][e, :]
            return 0
        lax.fori_loop(0, C, _edge, 0)

    issue_idx(0, 0)
    issue_idx(1, 1)
    wait_idx(0)
    issue_data(0, 0, 0)

    def _quad(i, _):
        for j in range(4):
            g = 4 * i + j
            bd = j % 2
            bi = j

            @pl.when(g >= 2)
            def _():
                wait_scatter((j - 2) % 4, bd)
            wait_idx((j + 1) % 4)
            issue_data(g + 1, (j + 1) % 4, 1 - bd)
            wait_data(bi, bd)

            @pl.when(g + 2 < NCHUNK)
            def _():
                issue_idx(g + 2, (j + 2) % 4)
            compute(bd)
            issue_scatter(bi, bd)
        return 0
    lax.fori_loop(0, (NCHUNK - 1) // 4, _quad, 0)

    wait_scatter(2, 0)
    wait_data(0, 0)
    compute(0)
    issue_scatter(0, 0)
    wait_scatter(3, 1)
    wait_scatter(0, 0)
    plsc.subcore_barrier()

    row0 = sid * ROWS_PER_TILE
    pltpu.sync_copy(acc_sh.at[pl.ds(row0, ROWS_PER_TILE)], slab_v)
    pltpu.sync_copy(slab_v, out_hbm.at[pl.ds(cid * NPAD + row0, ROWS_PER_TILE)])


def _final_add_body(p_ref, out_ref):
    out_ref[...] = p_ref[:N, :] + p_ref[NPAD:NPAD + N, :]


def kernel(chi, sph_ij, x, w_ij, idx_i, phi_r_cut, phi_chi_cut, idx_j, Wq, Wk):
    del chi
    bq = jnp.zeros((F, F), jnp.float32)
    bk = jnp.zeros((F, F), jnp.float32)
    for h in range(H):
        s = slice(h * FH, (h + 1) * FH)
        bq = bq.at[s, s].set(Wq[h].T)
        bk = bk.at[s, s].set(Wk[h].T)

    q, k = pl.pallas_call(
        _prep_body,
        out_shape=(jax.ShapeDtypeStruct((N, F), jnp.float32),
                   jax.ShapeDtypeStruct((N, F), jnp.float32)),
    )(x, bq, bk)

    mesh = plsc.VectorSubcoreMesh(core_axis_name="c", subcore_axis_name="s")
    partials = pl.kernel(
        _sc_edges_body,
        mesh=mesh,
        compiler_params=pltpu.CompilerParams(use_tc_tiling_on_sc=False),
        out_type=jax.ShapeDtypeStruct((NC * NPAD, M), jnp.float32),
        scratch_types=[
            pltpu.VMEM_SHARED((NPAD, M), jnp.float32),
            [pltpu.VMEM((C,), jnp.int32) for _ in range(4)],
            [pltpu.VMEM((C,), jnp.int32) for _ in range(4)],
            [pltpu.VMEM((C, F), jnp.float32) for _ in range(2)],
            [pltpu.VMEM((C, F), jnp.float32) for _ in range(2)],
            [pltpu.VMEM((C, F), jnp.float32) for _ in range(2)],
            [pltpu.VMEM((C, M), jnp.float32) for _ in range(2)],
            [pltpu.VMEM((C,), jnp.float32) for _ in range(2)],
            [pltpu.VMEM((C,), jnp.float32) for _ in range(2)],
            [pltpu.VMEM((C + 16,), jnp.float32) for _ in range(2)],
            [pltpu.VMEM((C, M), jnp.float32) for _ in range(2)],
            pltpu.VMEM((ROWS_PER_TILE, M), jnp.float32),
            [pltpu.SemaphoreType.DMA for _ in range(4)],
            [pltpu.SemaphoreType.DMA for _ in range(2)],
            [pltpu.SemaphoreType.DMA for _ in range(2)],
        ],
    )(q, k, w_ij, sph_ij, phi_r_cut.reshape(P), phi_chi_cut, idx_i, idx_j)

    chi_out = pl.pallas_call(
        _final_add_body,
        out_shape=jax.ShapeDtypeStruct((N, M), jnp.float32),
    )(partials)
    return chi_out


# speedup vs baseline: 1.2042x; 1.1501x over previous
import functools
import math

import jax
import jax.numpy as jnp
from jax import lax
from jax.experimental import pallas as pl
from jax.experimental.pallas import tpu as pltpu
from jax.experimental.pallas import tpu_sc as plsc

N = 10000
P = 320000
F = 128
H = 4
FH = F // H
M = 16

NC = 2
NS = 16
NW = NC * NS
EPW = P // NW
C = 80
NCHUNK = EPW // C
NPAD = 10240
ROWS_PER_TILE = NPAD // NS


def _prep_body(x_ref, bq_ref, bk_ref, q_ref, k_ref):
    xv = x_ref[...]
    q_ref[...] = jnp.dot(xv, bq_ref[...], preferred_element_type=jnp.float32)
    k_ref[...] = jnp.dot(xv, bk_ref[...], preferred_element_type=jnp.float32)


_RSQRT_FH = 1.0 / math.sqrt(FH)


def _sph128_body(sph_ref, out_ref):
    v = sph_ref[...]
    out_ref[...] = jnp.concatenate([v[s::8, :] for s in range(8)], axis=1)


_GDN = lax.GatherDimensionNumbers(offset_dims=(), collapsed_slice_dims=(0,),
                                  start_index_map=(0,))


def _shuffle(v, perm):
    return lax.gather(v, perm[:, None], dimension_numbers=_GDN,
                      slice_sizes=(1,), mode=lax.GatherScatterMode.PROMISE_IN_BOUNDS)


def _splat_sum(v, iota):
    for sft in (1, 2, 4, 8):
        v = v + _shuffle(v, jnp.bitwise_xor(iota, sft))
    return v


def _rep_from_heads(hs, iota):
    lo8 = iota < 8
    a = [h + _shuffle(h, jnp.bitwise_xor(iota, 8)) for h in hs]
    ab = jnp.where(lo8, a[0], a[1])
    cd = jnp.where(lo8, a[2], a[3])
    for sft in (4, 2, 1):
        perm = jnp.bitwise_xor(iota, sft)
        ab = ab + _shuffle(ab, perm)
        cd = cd + _shuffle(cd, perm)
    pat_ab = jnp.where(iota < 1, 0, 8)
    pat_cd = jnp.where(iota < 9, 0, 8)
    return jnp.where(iota < 4, _shuffle(ab, pat_ab), _shuffle(cd, pat_cd))
def _sc_edges_body(q_hbm, k_hbm, w_hbm, sph_hbm, phir_hbm, phic_hbm,
                   idxi_hbm, idxj_hbm,
                   out_hbm,
                   acc_sh,
                   idxi_v, idxj_v, q_v, k_v, w_v, sph_v, phir_v, phic_v,
                   scl_v, contrib_v,
                   slab_v, sem_idx, sem_dat, sem_sc):
    cid = lax.axis_index("c")
    sid = lax.axis_index("s")
    wid = sid * NC + cid

    def _zero_row(r, _):
        slab_v[r, :] = jnp.zeros((M,), jnp.float32)
        return 0
    lax.fori_loop(0, ROWS_PER_TILE, _zero_row, 0)
    pltpu.sync_copy(slab_v, acc_sh.at[pl.ds(sid * ROWS_PER_TILE, ROWS_PER_TILE)])
    plsc.subcore_barrier()

    iota = lax.iota(jnp.int32, 16)

    def _base(g):
        return wid * EPW + g * C

    def issue_idx(g, bi):
        pltpu.async_copy(idxi_hbm.at[pl.ds(_base(g), C)], idxi_v[bi], sem_idx[bi])
        pltpu.async_copy(idxj_hbm.at[pl.ds(_base(g), C)], idxj_v[bi], sem_idx[bi])

    def wait_idx(bi):
        pltpu.make_async_copy(idxi_hbm.at[pl.ds(0, C)], idxi_v[bi], sem_idx[bi]).wait()
        pltpu.make_async_copy(idxj_hbm.at[pl.ds(0, C)], idxj_v[bi], sem_idx[bi]).wait()

    def issue_data(g, bi, bd):
        pltpu.async_copy(q_hbm.at[idxi_v[bi]], q_v[bd], sem_dat[bd])
        pltpu.async_copy(k_hbm.at[idxj_v[bi]], k_v[bd], sem_dat[bd])
        pltpu.async_copy(w_hbm.at[pl.ds(_base(g), C)], w_v[bd], sem_dat[bd])
        pltpu.async_copy(sph_hbm.at[pl.ds(_base(g) * M // F, C * M // F)],
                         sph_v[bd], sem_dat[bd])
        pltpu.async_copy(phir_hbm.at[pl.ds(_base(g), C)], phir_v[bd],
                         sem_dat[bd])
        pltpu.async_copy(phic_hbm.at[pl.ds(_base(g), C)], phic_v[bd], sem_dat[bd])

    def wait_data(bi, bd):
        pltpu.make_async_copy(q_hbm.at[idxi_v[bi]], q_v[bd], sem_dat[bd]).wait()
        pltpu.make_async_copy(k_hbm.at[idxj_v[bi]], k_v[bd], sem_dat[bd]).wait()
        pltpu.make_async_copy(w_hbm.at[pl.ds(0, C)], w_v[bd], sem_dat[bd]).wait()
        pltpu.make_async_copy(sph_hbm.at[pl.ds(0, C * M // F)], sph_v[bd],
                              sem_dat[bd]).wait()
        pltpu.make_async_copy(phir_hbm.at[pl.ds(0, C)], phir_v[bd],
                              sem_dat[bd]).wait()
        pltpu.make_async_copy(phic_hbm.at[pl.ds(0, C)], phic_v[bd], sem_dat[bd]).wait()

    def issue_scatter(bi, bd):
        pltpu.async_copy(contrib_v[bd], acc_sh.at[idxi_v[bi]], sem_sc[bd], add=True)

    def wait_scatter(bi, bd):
        pltpu.make_async_copy(contrib_v[bd], acc_sh.at[idxi_v[bi]], sem_sc[bd]).wait()

    zero16 = jnp.zeros((16,), jnp.int32)

    def compute(bd):
        for t in range(C // 16):
            s = pl.ds(16 * t, 16)
            scl_v[bd][s] = (phir_v[bd][s] + phic_v[bd][s]) * _RSQRT_FH

        def _one(e):
            hs = []
            for hh in range(H):
                p0 = (q_v[bd][e, pl.ds(32 * hh, 16)]
                      * w_v[bd][e, pl.ds(32 * hh, 16)]
                      * k_v[bd][e, pl.ds(32 * hh, 16)])
                p1 = (q_v[bd][e, pl.ds(32 * hh + 16, 16)]
                      * w_v[bd][e, pl.ds(32 * hh + 16, 16)]
                      * k_v[bd][e, pl.ds(32 * hh + 16, 16)])
                hs.append(_splat_sum(p0 + p1, iota))
            rep = jnp.where(iota < 1, hs[0],
                            jnp.where(iota < 4, hs[1],
                                      jnp.where(iota < 9, hs[2], hs[3])))
            scale = _shuffle(scl_v[bd][pl.ds(e, 16)], zero16)
            sph_row = sph_v[bd][e // 8, pl.ds((e % 8) * M, 16)]
            return rep * scale * sph_row

        def _edge_pair(i, _):
            e = 2 * i
            r0 = _one(e)
            r1 = _one(e + 1)
            contrib_v[bd][e, :] = r0
            contrib_v[bd][e + 1, :] = r1
            return 0
        lax.fori_loop(0, C // 2, _edge_pair, 0)

    issue_idx(0, 0)
    issue_idx(1, 1)
    wait_idx(0)
    issue_data(0, 0, 0)

    def _quad(i, _):
        for j in range(4):
            g = 4 * i + j
            bd = j % 2
            bi = j

            @pl.when(g >= 2)
            def _():
                wait_scatter((j - 2) % 4, bd)
            wait_idx((j + 1) % 4)
            issue_data(g + 1, (j + 1) % 4, 1 - bd)
            wait_data(bi, bd)

            @pl.when(g + 2 < NCHUNK)
            def _():
                issue_idx(g + 2, (j + 2) % 4)
            compute(bd)
            issue_scatter(bi, bd)
        return 0
    lax.fori_loop(0, (NCHUNK - 1) // 4, _quad, 0)

    wait_scatter(2, 0)
    wait_data(0, 0)
    compute(0)
    issue_scatter(0, 0)
    wait_scatter(3, 1)
    wait_scatter(0, 0)
    plsc.subcore_barrier()

    row0 = sid * ROWS_PER_TILE
    pltpu.sync_copy(acc_sh.at[pl.ds(row0, ROWS_PER_TILE)], slab_v)
    pltpu.sync_copy(slab_v, out_hbm.at[pl.ds(cid * NPAD + row0, ROWS_PER_TILE)])


def _final_add_body(p_ref, out_ref):
    out_ref[...] = p_ref[:N, :] + p_ref[NPAD:NPAD + N, :]


def kernel(chi, sph_ij, x, w_ij, idx_i, phi_r_cut, phi_chi_cut, idx_j, Wq, Wk):
    del chi
    bq = jnp.zeros((F, F), jnp.float32)
    bk = jnp.zeros((F, F), jnp.float32)
    for h in range(H):
        s = slice(h * FH, (h + 1) * FH)
        bq = bq.at[s, s].set(Wq[h].T)
        bk = bk.at[s, s].set(Wk[h].T)

    q, k = pl.pallas_call(
        _prep_body,
        out_shape=(jax.ShapeDtypeStruct((N, F), jnp.float32),
                   jax.ShapeDtypeStruct((N, F), jnp.float32)),
    )(x, bq, bk)

    sph128 = sph_ij.reshape(P * M // F, F)

    mesh = plsc.VectorSubcoreMesh(core_axis_name="c", subcore_axis_name="s")
    partials = pl.kernel(
        _sc_edges_body,
        mesh=mesh,
        compiler_params=pltpu.CompilerParams(use_tc_tiling_on_sc=False),
        out_type=jax.ShapeDtypeStruct((NC * NPAD, M), jnp.float32),
        scratch_types=[
            pltpu.VMEM_SHARED((NPAD, M), jnp.float32),
            [pltpu.VMEM((C,), jnp.int32) for _ in range(4)],
            [pltpu.VMEM((C,), jnp.int32) for _ in range(4)],
            [pltpu.VMEM((C, F), jnp.float32) for _ in range(2)],
            [pltpu.VMEM((C, F), jnp.float32) for _ in range(2)],
            [pltpu.VMEM((C, F), jnp.float32) for _ in range(2)],
            [pltpu.VMEM((C * M // F, F), jnp.float32) for _ in range(2)],
            [pltpu.VMEM((C,), jnp.float32) for _ in range(2)],
            [pltpu.VMEM((C,), jnp.float32) for _ in range(2)],
            [pltpu.VMEM((C + 16,), jnp.float32) for _ in range(2)],
            [pltpu.VMEM((C, M), jnp.float32) for _ in range(2)],
            pltpu.VMEM((ROWS_PER_TILE, M), jnp.float32),
            [pltpu.SemaphoreType.DMA for _ in range(4)],
            [pltpu.SemaphoreType.DMA for _ in range(2)],
            [pltpu.SemaphoreType.DMA for _ in range(2)],
        ],
    )(q, k, w_ij, sph128, phi_r_cut.reshape(P), phi_chi_cut, idx_i, idx_j)

    chi_out = pl.pallas_call(
        _final_add_body,
        out_shape=jax.ShapeDtypeStruct((N, M), jnp.float32),
    )(partials)
    return chi_out


# speedup vs baseline: 1.2673x; 1.0524x over previous
import functools
import math

import jax
import jax.numpy as jnp
from jax import lax
from jax.experimental import pallas as pl
from jax.experimental.pallas import tpu as pltpu
from jax.experimental.pallas import tpu_sc as plsc

N = 10000
P = 320000
F = 128
H = 4
FH = F // H
M = 16

NC = 2
NS = 16
NW = NC * NS
EPW = P // NW
C = 80
NCHUNK = EPW // C
NPAD = 10240
ROWS_PER_TILE = NPAD // NS


def _prep_body(x_ref, bq_ref, bk_ref, q_ref, k_ref):
    xv = x_ref[...]
    q_ref[...] = jnp.dot(xv, bq_ref[...], preferred_element_type=jnp.float32)
    k_ref[...] = jnp.dot(xv, bk_ref[...], preferred_element_type=jnp.float32)


_RSQRT_FH = 1.0 / math.sqrt(FH)


def _sph128_body(sph_ref, out_ref):
    v = sph_ref[...]
    out_ref[...] = jnp.concatenate([v[s::8, :] for s in range(8)], axis=1)


_GDN = lax.GatherDimensionNumbers(offset_dims=(), collapsed_slice_dims=(0,),
                                  start_index_map=(0,))


def _shuffle(v, perm):
    return lax.gather(v, perm[:, None], dimension_numbers=_GDN,
                      slice_sizes=(1,), mode=lax.GatherScatterMode.PROMISE_IN_BOUNDS)


def _splat_sum(v, iota):
    for sft in (1, 2, 4, 8):
        v = v + _shuffle(v, jnp.bitwise_xor(iota, sft))
    return v


def _rep_from_heads(hs, iota):
    lo8 = iota < 8
    a = [h + _shuffle(h, jnp.bitwise_xor(iota, 8)) for h in hs]
    ab = jnp.where(lo8, a[0], a[1])
    cd = jnp.where(lo8, a[2], a[3])
    for sft in (4, 2, 1):
        perm = jnp.bitwise_xor(iota, sft)
        ab = ab + _shuffle(ab, perm)
        cd = cd + _shuffle(cd, perm)
    pat_ab = jnp.where(iota < 1, 0, 8)
    pat_cd = jnp.where(iota < 9, 0, 8)
    return jnp.where(iota < 4, _shuffle(ab, pat_ab), _shuffle(cd, pat_cd))
def _sc_edges_body(q_hbm, k_hbm, w_hbm, sph_hbm, phir_hbm, phic_hbm,
                   idxi_hbm, idxj_hbm,
                   out_hbm,
                   acc_sh,
                   idxi_v, idxj_v, q_v, k_v, w_v, sph_v, phir_v, phic_v,
                   scl_v, contrib_v,
                   slab_v, sem_idx, sem_dat, sem_sc):
    cid = lax.axis_index("c")
    sid = lax.axis_index("s")
    wid = sid * NC + cid

    def _zero_row(r, _):
        slab_v[r, :] = jnp.zeros((M,), jnp.float32)
        return 0
    lax.fori_loop(0, ROWS_PER_TILE, _zero_row, 0)
    pltpu.sync_copy(slab_v, acc_sh.at[pl.ds(sid * ROWS_PER_TILE, ROWS_PER_TILE)])
    plsc.subcore_barrier()

    iota = lax.iota(jnp.int32, 16)

    def _base(g):
        return wid * EPW + g * C

    def issue_idx(g, bi):
        pltpu.async_copy(idxi_hbm.at[pl.ds(_base(g), C)], idxi_v[bi], sem_idx[bi])
        pltpu.async_copy(idxj_hbm.at[pl.ds(_base(g), C)], idxj_v[bi], sem_idx[bi])

    def wait_idx(bi):
        pltpu.make_async_copy(idxi_hbm.at[pl.ds(0, C)], idxi_v[bi], sem_idx[bi]).wait()
        pltpu.make_async_copy(idxj_hbm.at[pl.ds(0, C)], idxj_v[bi], sem_idx[bi]).wait()

    def issue_data(g, bi, bd):
        pltpu.async_copy(q_hbm.at[idxi_v[bi]], q_v[bd], sem_dat[bd])
        pltpu.async_copy(k_hbm.at[idxj_v[bi]], k_v[bd], sem_dat[bd])
        pltpu.async_copy(w_hbm.at[pl.ds(_base(g), C)], w_v[bd], sem_dat[bd])
        pltpu.async_copy(sph_hbm.at[pl.ds(_base(g) * M // F, C * M // F)],
                         sph_v[bd], sem_dat[bd])
        pltpu.async_copy(phir_hbm.at[pl.ds(_base(g), C)], phir_v[bd],
                         sem_dat[bd])
        pltpu.async_copy(phic_hbm.at[pl.ds(_base(g), C)], phic_v[bd], sem_dat[bd])

    def wait_data(bi, bd):
        pltpu.make_async_copy(q_hbm.at[idxi_v[bi]], q_v[bd], sem_dat[bd]).wait()
        pltpu.make_async_copy(k_hbm.at[idxj_v[bi]], k_v[bd], sem_dat[bd]).wait()
        pltpu.make_async_copy(w_hbm.at[pl.ds(0, C)], w_v[bd], sem_dat[bd]).wait()
        pltpu.make_async_copy(sph_hbm.at[pl.ds(0, C * M // F)], sph_v[bd],
                              sem_dat[bd]).wait()
        pltpu.make_async_copy(phir_hbm.at[pl.ds(0, C)], phir_v[bd],
                              sem_dat[bd]).wait()
        pltpu.make_async_copy(phic_hbm.at[pl.ds(0, C)], phic_v[bd], sem_dat[bd]).wait()

    def issue_scatter(bi, bd):
        pltpu.async_copy(contrib_v[bd], acc_sh.at[idxi_v[bi]], sem_sc[bd], add=True)

    def wait_scatter(bi, bd):
        pltpu.make_async_copy(contrib_v[bd], acc_sh.at[idxi_v[bi]], sem_sc[bd]).wait()

    zero16 = jnp.zeros((16,), jnp.int32)

    def compute(bd):
        for t in range(C // 16):
            s = pl.ds(16 * t, 16)
            scl_v[bd][s] = (phir_v[bd][s] + phic_v[bd][s]) * _RSQRT_FH

        def _one(e):
            hs = []
            for hh in range(H):
                p0 = (q_v[bd][e, pl.ds(32 * hh, 16)]
                      * w_v[bd][e, pl.ds(32 * hh, 16)]
                      * k_v[bd][e, pl.ds(32 * hh, 16)])
                p1 = (q_v[bd][e, pl.ds(32 * hh + 16, 16)]
                      * w_v[bd][e, pl.ds(32 * hh + 16, 16)]
                      * k_v[bd][e, pl.ds(32 * hh + 16, 16)])
                hs.append(_splat_sum(p0 + p1, iota))
            rep = jnp.where(iota < 1, hs[0],
                            jnp.where(iota < 4, hs[1],
                                      jnp.where(iota < 9, hs[2], hs[3])))
            scale = _shuffle(scl_v[bd][pl.ds(e, 16)], zero16)
            sph_row = sph_v[bd][e // 8, pl.ds((e % 8) * M, 16)]
            return rep * scale * sph_row

        def _edge_quad(i, _):
            e = 4 * i
            rs = [_one(e + j) for j in range(4)]
            for j in range(4):
                contrib_v[bd][e + j, :] = rs[j]
            return 0
        lax.fori_loop(0, C // 4, _edge_quad, 0)

    issue_idx(0, 0)
    issue_idx(1, 1)
    wait_idx(0)
    issue_data(0, 0, 0)

    def _quad(i, _):
        for j in range(4):
            g = 4 * i + j
            bd = j % 2
            bi = j

            @pl.when(g >= 2)
            def _():
                wait_scatter((j - 2) % 4, bd)
            wait_idx((j + 1) % 4)
            issue_data(g + 1, (j + 1) % 4, 1 - bd)
            wait_data(bi, bd)

            @pl.when(g + 2 < NCHUNK)
            def _():
                issue_idx(g + 2, (j + 2) % 4)
            compute(bd)
            issue_scatter(bi, bd)
        return 0
    lax.fori_loop(0, (NCHUNK - 1) // 4, _quad, 0)

    wait_scatter(2, 0)
    wait_data(0, 0)
    compute(0)
    issue_scatter(0, 0)
    wait_scatter(3, 1)
    wait_scatter(0, 0)
    plsc.subcore_barrier()

    row0 = sid * ROWS_PER_TILE
    pltpu.sync_copy(acc_sh.at[pl.ds(row0, ROWS_PER_TILE)], slab_v)
    pltpu.sync_copy(slab_v, out_hbm.at[pl.ds(cid * NPAD + row0, ROWS_PER_TILE)])


def _final_add_body(p_ref, out_ref):
    out_ref[...] = p_ref[:N, :] + p_ref[NPAD:NPAD + N, :]


def kernel(chi, sph_ij, x, w_ij, idx_i, phi_r_cut, phi_chi_cut, idx_j, Wq, Wk):
    del chi
    bq = jnp.zeros((F, F), jnp.float32)
    bk = jnp.zeros((F, F), jnp.float32)
    for h in range(H):
        s = slice(h * FH, (h + 1) * FH)
        bq = bq.at[s, s].set(Wq[h].T)
        bk = bk.at[s, s].set(Wk[h].T)

    q, k = pl.pallas_call(
        _prep_body,
        out_shape=(jax.ShapeDtypeStruct((N, F), jnp.float32),
                   jax.ShapeDtypeStruct((N, F), jnp.float32)),
    )(x, bq, bk)

    sph128 = sph_ij.reshape(P * M // F, F)

    mesh = plsc.VectorSubcoreMesh(core_axis_name="c", subcore_axis_name="s")
    partials = pl.kernel(
        _sc_edges_body,
        mesh=mesh,
        compiler_params=pltpu.CompilerParams(use_tc_tiling_on_sc=False),
        out_type=jax.ShapeDtypeStruct((NC * NPAD, M), jnp.float32),
        scratch_types=[
            pltpu.VMEM_SHARED((NPAD, M), jnp.float32),
            [pltpu.VMEM((C,), jnp.int32) for _ in range(4)],
            [pltpu.VMEM((C,), jnp.int32) for _ in range(4)],
            [pltpu.VMEM((C, F), jnp.float32) for _ in range(2)],
            [pltpu.VMEM((C, F), jnp.float32) for _ in range(2)],
            [pltpu.VMEM((C, F), jnp.float32) for _ in range(2)],
            [pltpu.VMEM((C * M // F, F), jnp.float32) for _ in range(2)],
            [pltpu.VMEM((C,), jnp.float32) for _ in range(2)],
            [pltpu.VMEM((C,), jnp.float32) for _ in range(2)],
            [pltpu.VMEM((C + 16,), jnp.float32) for _ in range(2)],
            [pltpu.VMEM((C, M), jnp.float32) for _ in range(2)],
            pltpu.VMEM((ROWS_PER_TILE, M), jnp.float32),
            [pltpu.SemaphoreType.DMA for _ in range(4)],
            [pltpu.SemaphoreType.DMA for _ in range(2)],
            [pltpu.SemaphoreType.DMA for _ in range(2)],
        ],
    )(q, k, w_ij, sph128, phi_r_cut.reshape(P), phi_chi_cut, idx_i, idx_j)

    chi_out = pl.pallas_call(
        _final_add_body,
        out_shape=jax.ShapeDtypeStruct((N, M), jnp.float32),
    )(partials)
    return chi_out


# speedup vs baseline: 1.2976x; 1.0239x over previous
import functools
import math

import jax
import jax.numpy as jnp
from jax import lax
from jax.experimental import pallas as pl
from jax.experimental.pallas import tpu as pltpu
from jax.experimental.pallas import tpu_sc as plsc

N = 10000
P = 320000
F = 128
H = 4
FH = F // H
M = 16

NC = 2
NS = 16
NW = NC * NS
EPW = P // NW
C = 80
NCHUNK = EPW // C
NPAD = 10240
ROWS_PER_TILE = NPAD // NS


def _prep_body(x_ref, bq_ref, bk_ref, q_ref, k_ref):
    xv = x_ref[...]
    q_ref[...] = jnp.dot(xv, bq_ref[...], preferred_element_type=jnp.float32)
    k_ref[...] = jnp.dot(xv, bk_ref[...], preferred_element_type=jnp.float32)


_RSQRT_FH = 1.0 / math.sqrt(FH)


def _sph128_body(sph_ref, out_ref):
    v = sph_ref[...]
    out_ref[...] = jnp.concatenate([v[s::8, :] for s in range(8)], axis=1)


_GDN = lax.GatherDimensionNumbers(offset_dims=(), collapsed_slice_dims=(0,),
                                  start_index_map=(0,))


def _shuffle(v, perm):
    return lax.gather(v, perm[:, None], dimension_numbers=_GDN,
                      slice_sizes=(1,), mode=lax.GatherScatterMode.PROMISE_IN_BOUNDS)


def _splat_sum(v, iota):
    for sft in (1, 2, 4, 8):
        v = v + _shuffle(v, jnp.bitwise_xor(iota, sft))
    return v


def _rep_from_heads(hs, iota):
    lo8 = iota < 8
    a = [h + _shuffle(h, jnp.bitwise_xor(iota, 8)) for h in hs]
    ab = jnp.where(lo8, a[0], a[1])
    cd = jnp.where(lo8, a[2], a[3])
    for sft in (4, 2, 1):
        perm = jnp.bitwise_xor(iota, sft)
        ab = ab + _shuffle(ab, perm)
        cd = cd + _shuffle(cd, perm)
    pat_ab = jnp.where(iota < 1, 0, 8)
    pat_cd = jnp.where(iota < 9, 0, 8)
    return jnp.where(iota < 4, _shuffle(ab, pat_ab), _shuffle(cd, pat_cd))
def _sc_edges_body(q_hbm, k_hbm, w_hbm, sph_hbm, phir_hbm, phic_hbm,
                   idxi_hbm, idxj_hbm,
                   out_hbm,
                   acc_sh,
                   idxi_v, idxj_v, q_v, k_v, w_v, sph_v, phir_v, phic_v,
                   scl_v, contrib_v,
                   slab_v, sem_idx, sem_dat, sem_sc):
    cid = lax.axis_index("c")
    sid = lax.axis_index("s")
    wid = sid * NC + cid

    def _zero_row(r, _):
        slab_v[r, :] = jnp.zeros((M,), jnp.float32)
        return 0
    lax.fori_loop(0, ROWS_PER_TILE, _zero_row, 0)
    pltpu.sync_copy(slab_v, acc_sh.at[pl.ds(sid * ROWS_PER_TILE, ROWS_PER_TILE)])
    plsc.subcore_barrier()

    iota = lax.iota(jnp.int32, 16)

    def _base(g):
        return wid * EPW + g * C

    def issue_idx(g, bi):
        pltpu.async_copy(idxi_hbm.at[pl.ds(_base(g), C)], idxi_v[bi], sem_idx[bi])
        pltpu.async_copy(idxj_hbm.at[pl.ds(_base(g), C)], idxj_v[bi], sem_idx[bi])

    def wait_idx(bi):
        pltpu.make_async_copy(idxi_hbm.at[pl.ds(0, C)], idxi_v[bi], sem_idx[bi]).wait()
        pltpu.make_async_copy(idxj_hbm.at[pl.ds(0, C)], idxj_v[bi], sem_idx[bi]).wait()

    def issue_data(g, bi, bd):
        pltpu.async_copy(q_hbm.at[idxi_v[bi]], q_v[bd], sem_dat[bd])
        pltpu.async_copy(k_hbm.at[idxj_v[bi]], k_v[bd], sem_dat[bd])
        pltpu.async_copy(w_hbm.at[pl.ds(_base(g), C)], w_v[bd], sem_dat[bd])
        pltpu.async_copy(sph_hbm.at[pl.ds(_base(g) * M // F, C * M // F)],
                         sph_v[bd], sem_dat[bd])
        pltpu.async_copy(phir_hbm.at[pl.ds(_base(g), C)], phir_v[bd],
                         sem_dat[bd])
        pltpu.async_copy(phic_hbm.at[pl.ds(_base(g), C)], phic_v[bd], sem_dat[bd])

    def wait_data(bi, bd):
        pltpu.make_async_copy(q_hbm.at[idxi_v[bi]], q_v[bd], sem_dat[bd]).wait()
        pltpu.make_async_copy(k_hbm.at[idxj_v[bi]], k_v[bd], sem_dat[bd]).wait()
        pltpu.make_async_copy(w_hbm.at[pl.ds(0, C)], w_v[bd], sem_dat[bd]).wait()
        pltpu.make_async_copy(sph_hbm.at[pl.ds(0, C * M // F)], sph_v[bd],
                              sem_dat[bd]).wait()
        pltpu.make_async_copy(phir_hbm.at[pl.ds(0, C)], phir_v[bd],
                              sem_dat[bd]).wait()
        pltpu.make_async_copy(phic_hbm.at[pl.ds(0, C)], phic_v[bd], sem_dat[bd]).wait()

    def issue_scatter(bi, bd):
        pltpu.async_copy(contrib_v[bd], acc_sh.at[idxi_v[bi]], sem_sc[bd], add=True)

    def wait_scatter(bi, bd):
        pltpu.make_async_copy(contrib_v[bd], acc_sh.at[idxi_v[bi]], sem_sc[bd]).wait()

    zero16 = jnp.zeros((16,), jnp.int32)

    def compute(bd):
        for t in range(C // 16):
            s = pl.ds(16 * t, 16)
            scl_v[bd][s] = (phir_v[bd][s] + phic_v[bd][s]) * _RSQRT_FH

        def _one(e):
            hs = []
            for hh in range(H):
                p0 = (q_v[bd][e, pl.ds(32 * hh, 16)]
                      * w_v[bd][e, pl.ds(32 * hh, 16)]
                      * k_v[bd][e, pl.ds(32 * hh, 16)])
                p1 = (q_v[bd][e, pl.ds(32 * hh + 16, 16)]
                      * w_v[bd][e, pl.ds(32 * hh + 16, 16)]
                      * k_v[bd][e, pl.ds(32 * hh + 16, 16)])
                hs.append(_splat_sum(p0 + p1, iota))
            rep = jnp.where(iota < 1, hs[0],
                            jnp.where(iota < 4, hs[1],
                                      jnp.where(iota < 9, hs[2], hs[3])))
            scale = _shuffle(scl_v[bd][pl.ds(e, 16)], zero16)
            sph_row = sph_v[bd][e // 8, pl.ds((e % 8) * M, 16)]
            return rep * scale * sph_row

        def _edge_quad(i, _):
            e = 8 * i
            rs = [_one(e + j) for j in range(8)]
            for j in range(8):
                contrib_v[bd][e + j, :] = rs[j]
            return 0
        lax.fori_loop(0, C // 8, _edge_quad, 0)

    issue_idx(0, 0)
    issue_idx(1, 1)
    wait_idx(0)
    issue_data(0, 0, 0)

    def _quad(i, _):
        for j in range(4):
            g = 4 * i + j
            bd = j % 2
            bi = j

            @pl.when(g >= 2)
            def _():
                wait_scatter((j - 2) % 4, bd)
            wait_idx((j + 1) % 4)
            issue_data(g + 1, (j + 1) % 4, 1 - bd)
            wait_data(bi, bd)

            @pl.when(g + 2 < NCHUNK)
            def _():
                issue_idx(g + 2, (j + 2) % 4)
            compute(bd)
            issue_scatter(bi, bd)
        return 0
    lax.fori_loop(0, (NCHUNK - 1) // 4, _quad, 0)

    wait_scatter(2, 0)
    wait_data(0, 0)
    compute(0)
    issue_scatter(0, 0)
    wait_scatter(3, 1)
    wait_scatter(0, 0)
    plsc.subcore_barrier()

    row0 = sid * ROWS_PER_TILE
    pltpu.sync_copy(acc_sh.at[pl.ds(row0, ROWS_PER_TILE)], slab_v)
    pltpu.sync_copy(slab_v, out_hbm.at[pl.ds(cid * NPAD + row0, ROWS_PER_TILE)])


def _final_add_body(p_ref, out_ref):
    out_ref[...] = p_ref[:N, :] + p_ref[NPAD:NPAD + N, :]


def kernel(chi, sph_ij, x, w_ij, idx_i, phi_r_cut, phi_chi_cut, idx_j, Wq, Wk):
    del chi
    bq = jnp.zeros((F, F), jnp.float32)
    bk = jnp.zeros((F, F), jnp.float32)
    for h in range(H):
        s = slice(h * FH, (h + 1) * FH)
        bq = bq.at[s, s].set(Wq[h].T)
        bk = bk.at[s, s].set(Wk[h].T)

    q, k = pl.pallas_call(
        _prep_body,
        out_shape=(jax.ShapeDtypeStruct((N, F), jnp.float32),
                   jax.ShapeDtypeStruct((N, F), jnp.float32)),
    )(x, bq, bk)

    sph128 = sph_ij.reshape(P * M // F, F)

    mesh = plsc.VectorSubcoreMesh(core_axis_name="c", subcore_axis_name="s")
    partials = pl.kernel(
        _sc_edges_body,
        mesh=mesh,
        compiler_params=pltpu.CompilerParams(use_tc_tiling_on_sc=False),
        out_type=jax.ShapeDtypeStruct((NC * NPAD, M), jnp.float32),
        scratch_types=[
            pltpu.VMEM_SHARED((NPAD, M), jnp.float32),
            [pltpu.VMEM((C,), jnp.int32) for _ in range(4)],
            [pltpu.VMEM((C,), jnp.int32) for _ in range(4)],
            [pltpu.VMEM((C, F), jnp.float32) for _ in range(2)],
            [pltpu.VMEM((C, F), jnp.float32) for _ in range(2)],
            [pltpu.VMEM((C, F), jnp.float32) for _ in range(2)],
            [pltpu.VMEM((C * M // F, F), jnp.float32) for _ in range(2)],
            [pltpu.VMEM((C,), jnp.float32) for _ in range(2)],
            [pltpu.VMEM((C,), jnp.float32) for _ in range(2)],
            [pltpu.VMEM((C + 16,), jnp.float32) for _ in range(2)],
            [pltpu.VMEM((C, M), jnp.float32) for _ in range(2)],
            pltpu.VMEM((ROWS_PER_TILE, M), jnp.float32),
            [pltpu.SemaphoreType.DMA for _ in range(4)],
            [pltpu.SemaphoreType.DMA for _ in range(2)],
            [pltpu.SemaphoreType.DMA for _ in range(2)],
        ],
    )(q, k, w_ij, sph128, phi_r_cut.reshape(P), phi_chi_cut, idx_i, idx_j)

    chi_out = pl.pallas_call(
        _final_add_body,
        out_shape=jax.ShapeDtypeStruct((N, M), jnp.float32),
    )(partials)
    return chi_out


# speedup vs baseline: 1.4000x; 1.0789x over previous
import functools
import math

import jax
import jax.numpy as jnp
from jax import lax
from jax.experimental import pallas as pl
from jax.experimental.pallas import tpu as pltpu
from jax.experimental.pallas import tpu_sc as plsc

N = 10000
P = 320000
F = 128
H = 4
FH = F // H
M = 16

NC = 2
NS = 16
NW = NC * NS
EPW = P // NW
C = 80
NCHUNK = EPW // C
NPAD = 10240
ROWS_PER_TILE = NPAD // NS


def _prep_body(x_ref, bq_ref, bk_ref, q_ref, k_ref):
    xv = x_ref[...]
    q_ref[...] = jnp.dot(xv, bq_ref[...], preferred_element_type=jnp.float32)
    k_ref[...] = jnp.dot(xv, bk_ref[...], preferred_element_type=jnp.float32)


_RSQRT_FH = 1.0 / math.sqrt(FH)


def _sph128_body(sph_ref, out_ref):
    v = sph_ref[...]
    out_ref[...] = jnp.concatenate([v[s::8, :] for s in range(8)], axis=1)


_GDN = lax.GatherDimensionNumbers(offset_dims=(), collapsed_slice_dims=(0,),
                                  start_index_map=(0,))


def _shuffle(v, perm):
    return lax.gather(v, perm[:, None], dimension_numbers=_GDN,
                      slice_sizes=(1,), mode=lax.GatherScatterMode.PROMISE_IN_BOUNDS)


def _splat_sum(v, iota):
    for sft in (1, 2, 4, 8):
        v = v + _shuffle(v, jnp.bitwise_xor(iota, sft))
    return v


def _rep_from_heads(hs, iota):
    lo8 = iota < 8
    a = [h + _shuffle(h, jnp.bitwise_xor(iota, 8)) for h in hs]
    ab = jnp.where(lo8, a[0], a[1])
    cd = jnp.where(lo8, a[2], a[3])
    for sft in (4, 2, 1):
        perm = jnp.bitwise_xor(iota, sft)
        ab = ab + _shuffle(ab, perm)
        cd = cd + _shuffle(cd, perm)
    pat_ab = jnp.where(iota < 1, 0, 8)
    pat_cd = jnp.where(iota < 9, 0, 8)
    return jnp.where(iota < 4, _shuffle(ab, pat_ab), _shuffle(cd, pat_cd))
def _sc_edges_body(q_hbm, k_hbm, w_hbm, sph_hbm, phir_hbm, phic_hbm,
                   idxi_hbm, idxj_hbm,
                   out_hbm,
                   acc_sh,
                   idxi_v, idxj_v, q_v, k_v, w_v, sph_v, phir_v, phic_v,
                   scl_v, contrib_v,
                   slab_v, sem_idx, sem_dat, sem_sc):
    cid = lax.axis_index("c")
    sid = lax.axis_index("s")
    wid = sid * NC + cid

    def _zero_row(r, _):
        slab_v[r, :] = jnp.zeros((M,), jnp.float32)
        return 0
    lax.fori_loop(0, ROWS_PER_TILE, _zero_row, 0)
    pltpu.sync_copy(slab_v, acc_sh.at[pl.ds(sid * ROWS_PER_TILE, ROWS_PER_TILE)])
    plsc.subcore_barrier()

    iota = lax.iota(jnp.int32, 16)

    def _base(g):
        return wid * EPW + g * C

    def issue_idx(g, bi):
        pltpu.async_copy(idxi_hbm.at[pl.ds(_base(g), C)], idxi_v[bi], sem_idx[bi])
        pltpu.async_copy(idxj_hbm.at[pl.ds(_base(g), C)], idxj_v[bi], sem_idx[bi])

    def wait_idx(bi):
        pltpu.make_async_copy(idxi_hbm.at[pl.ds(0, C)], idxi_v[bi], sem_idx[bi]).wait()
        pltpu.make_async_copy(idxj_hbm.at[pl.ds(0, C)], idxj_v[bi], sem_idx[bi]).wait()

    def issue_data(g, bi, bd):
        pltpu.async_copy(q_hbm.at[idxi_v[bi]], q_v[bd], sem_dat[bd])
        pltpu.async_copy(k_hbm.at[idxj_v[bi]], k_v[bd], sem_dat[bd])
        pltpu.async_copy(w_hbm.at[pl.ds(_base(g), C)], w_v[bd], sem_dat[bd])
        pltpu.async_copy(sph_hbm.at[pl.ds(_base(g) * M // F, C * M // F)],
                         sph_v[bd], sem_dat[bd])
        pltpu.async_copy(phir_hbm.at[pl.ds(_base(g), C)], phir_v[bd],
                         sem_dat[bd])
        pltpu.async_copy(phic_hbm.at[pl.ds(_base(g), C)], phic_v[bd], sem_dat[bd])

    def wait_data(bi, bd):
        pltpu.make_async_copy(q_hbm.at[idxi_v[bi]], q_v[bd], sem_dat[bd]).wait()
        pltpu.make_async_copy(k_hbm.at[idxj_v[bi]], k_v[bd], sem_dat[bd]).wait()
        pltpu.make_async_copy(w_hbm.at[pl.ds(0, C)], w_v[bd], sem_dat[bd]).wait()
        pltpu.make_async_copy(sph_hbm.at[pl.ds(0, C * M // F)], sph_v[bd],
                              sem_dat[bd]).wait()
        pltpu.make_async_copy(phir_hbm.at[pl.ds(0, C)], phir_v[bd],
                              sem_dat[bd]).wait()
        pltpu.make_async_copy(phic_hbm.at[pl.ds(0, C)], phic_v[bd], sem_dat[bd]).wait()

    def issue_scatter(bi, bd):
        pltpu.async_copy(contrib_v[bd], acc_sh.at[idxi_v[bi]], sem_sc[bd], add=True)

    def wait_scatter(bi, bd):
        pltpu.make_async_copy(contrib_v[bd], acc_sh.at[idxi_v[bi]], sem_sc[bd]).wait()

    zero16 = jnp.zeros((16,), jnp.int32)

    def compute(bd):
        for t in range(C // 16):
            s = pl.ds(16 * t, 16)
            scl_v[bd][s] = (phir_v[bd][s] + phic_v[bd][s]) * _RSQRT_FH

        def _one(e):
            hs = []
            for hh in range(H):
                p0 = (q_v[bd][e, pl.ds(32 * hh, 16)]
                      * w_v[bd][e, pl.ds(32 * hh, 16)]
                      * k_v[bd][e, pl.ds(32 * hh, 16)])
                p1 = (q_v[bd][e, pl.ds(32 * hh + 16, 16)]
                      * w_v[bd][e, pl.ds(32 * hh + 16, 16)]
                      * k_v[bd][e, pl.ds(32 * hh + 16, 16)])
                hs.append(_splat_sum(p0 + p1, iota))
            rep = jnp.where(iota < 1, hs[0],
                            jnp.where(iota < 4, hs[1],
                                      jnp.where(iota < 9, hs[2], hs[3])))
            scale = _shuffle(scl_v[bd][pl.ds(e, 16)], zero16)
            sph_row = sph_v[bd][e // 8, pl.ds((e % 8) * M, 16)]
            return rep * scale * sph_row

        def _edge_quad(i, _):
            e = 8 * i
            rs = [_one(e + j) for j in range(8)]
            for j in range(8):
                contrib_v[bd][e + j, :] = rs[j]
            return 0
        lax.fori_loop(0, C // 8, _edge_quad, 0)

    issue_idx(0, 0)
    issue_idx(1, 1)
    issue_idx(2, 2)
    wait_idx(0)
    issue_data(0, 0, 0)
    wait_idx(1)
    issue_data(1, 1, 1)

    NSUP = (NCHUNK + 5) // 6

    def _super(i, _):
        for j in range(6):
            g = 6 * i + j
            bd = j % 3
            bi = j

            @pl.when(g >= 3)
            def _():
                wait_scatter((j - 3) % 6, bd)

            @pl.when(g + 3 < NCHUNK)
            def _():
                issue_idx(g + 3, (j + 3) % 6)

            @pl.when(g + 2 < NCHUNK)
            def _():
                wait_idx((j + 2) % 6)
                issue_data(g + 2, (j + 2) % 6, (j + 2) % 3)

            @pl.when(g < NCHUNK)
            def _():
                wait_data(bi, bd)
                compute(bd)
                issue_scatter(bi, bd)
        return 0
    lax.fori_loop(0, NSUP, _super, 0)

    wait_scatter(123 % 6, 123 % 3)
    wait_scatter(124 % 6, 124 % 3)
    plsc.subcore_barrier()

    row0 = sid * ROWS_PER_TILE
    pltpu.sync_copy(acc_sh.at[pl.ds(row0, ROWS_PER_TILE)], slab_v)
    pltpu.sync_copy(slab_v, out_hbm.at[pl.ds(cid * NPAD + row0, ROWS_PER_TILE)])


def _final_add_body(p_ref, out_ref):
    out_ref[...] = p_ref[:N, :] + p_ref[NPAD:NPAD + N, :]


def kernel(chi, sph_ij, x, w_ij, idx_i, phi_r_cut, phi_chi_cut, idx_j, Wq, Wk):
    del chi
    bq = jnp.zeros((F, F), jnp.float32)
    bk = jnp.zeros((F, F), jnp.float32)
    for h in range(H):
        s = slice(h * FH, (h + 1) * FH)
        bq = bq.at[s, s].set(Wq[h].T)
        bk = bk.at[s, s].set(Wk[h].T)

    q, k = pl.pallas_call(
        _prep_body,
        out_shape=(jax.ShapeDtypeStruct((N, F), jnp.float32),
                   jax.ShapeDtypeStruct((N, F), jnp.float32)),
    )(x, bq, bk)

    sph128 = sph_ij.reshape(P * M // F, F)

    mesh = plsc.VectorSubcoreMesh(core_axis_name="c", subcore_axis_name="s")
    partials = pl.kernel(
        _sc_edges_body,
        mesh=mesh,
        compiler_params=pltpu.CompilerParams(use_tc_tiling_on_sc=False),
        out_type=jax.ShapeDtypeStruct((NC * NPAD, M), jnp.float32),
        scratch_types=[
            pltpu.VMEM_SHARED((NPAD, M), jnp.float32),
            [pltpu.VMEM((C,), jnp.int32) for _ in range(6)],
            [pltpu.VMEM((C,), jnp.int32) for _ in range(6)],
            [pltpu.VMEM((C, F), jnp.float32) for _ in range(3)],
            [pltpu.VMEM((C, F), jnp.float32) for _ in range(3)],
            [pltpu.VMEM((C, F), jnp.float32) for _ in range(3)],
            [pltpu.VMEM((C * M // F, F), jnp.float32) for _ in range(3)],
            [pltpu.VMEM((C,), jnp.float32) for _ in range(3)],
            [pltpu.VMEM((C,), jnp.float32) for _ in range(3)],
            [pltpu.VMEM((C + 16,), jnp.float32) for _ in range(3)],
            [pltpu.VMEM((C, M), jnp.float32) for _ in range(3)],
            pltpu.VMEM((ROWS_PER_TILE, M), jnp.float32),
            [pltpu.SemaphoreType.DMA for _ in range(6)],
            [pltpu.SemaphoreType.DMA for _ in range(3)],
            [pltpu.SemaphoreType.DMA for _ in range(3)],
        ],
    )(q, k, w_ij, sph128, phi_r_cut.reshape(P), phi_chi_cut, idx_i, idx_j)

    chi_out = pl.pallas_call(
        _final_add_body,
        out_shape=jax.ShapeDtypeStruct((N, M), jnp.float32),
    )(partials)
    return chi_out
